# baseline (speedup 1.0000x reference)
import functools

import jax
import jax.numpy as jnp
from jax import lax
from jax.experimental import pallas as pl
from jax.experimental.pallas import tpu as pltpu
from jax.experimental.pallas import tpu_sc as plsc

N = 10000
E = 320000
D_IN = 128
E_IN = 16
HID = 128
NLAYERS = 4

NC = 2
NS = 16
NW = NC * NS
K = 128
NCHUNK = E // K
CPW = -(-NCHUNK // NW)
RPS = N // NS

_sc_mesh = plsc.VectorSubcoreMesh(
    core_axis_name="c", subcore_axis_name="s", num_cores=NC, num_subcores=NS)


def _zero_rows(rows_v, nrow, width):
  z = jnp.zeros((16,), jnp.float32)

  def body(i, _):
    for j in range(width // 16):
      rows_v[i, pl.ds(j * 16, 16)] = z
    return 0

  lax.fori_loop(0, nrow, body, 0, unroll=2)


def _make_spmm(l):
  @functools.partial(
      pl.kernel,
      out_type=jax.ShapeDtypeStruct((NC, N, HID), jnp.float32),
      mesh=_sc_mesh,
      scratch_types=[
          pltpu.VMEM((K,), jnp.int32),
          pltpu.VMEM((K,), jnp.int32),
          pltpu.VMEM((K,), jnp.float32),
          pltpu.VMEM((K, HID), jnp.float32),
          pltpu.VMEM((K, HID), jnp.float32),
          pltpu.VMEM_SHARED((N, HID), jnp.float32),
          pltpu.SemaphoreType.DMA,
          pltpu.SemaphoreType.DMA,
      ],
  )
  def _sc_spmm(hn_hbm, m2_hbm, src_hbm, dst_hbm, y4_hbm, out_hbm,
               src_v, dst_v, y_v, rows_v, m2_v, acc_sh, sem, sem2):
    cid = lax.axis_index("c")
    sid = lax.axis_index("s")
    wid = sid * NC + cid

    _zero_rows(rows_v, K, HID)
    for t in range(4):
      pltpu.sync_copy(rows_v.at[pl.ds(0, K)],
                      acc_sh.at[pl.ds(sid * 624 + t * K, K)])
    pltpu.sync_copy(rows_v.at[pl.ds(0, 112)],
                    acc_sh.at[pl.ds(sid * 624 + 4 * K, 112)])

    @pl.when(sid == 0)
    def _():
      pltpu.sync_copy(rows_v.at[pl.ds(0, 16)], acc_sh.at[pl.ds(16 * 624, 16)])

    plsc.subcore_barrier()

    def chunk(j, _):
      c = wid + j * NW

      @pl.when(c < NCHUNK)
      def _():
        base = c * K
        pltpu.sync_copy(src_hbm.at[pl.ds(base, K)], src_v)
        pltpu.sync_copy(dst_hbm.at[pl.ds(base, K)], dst_v)
        pltpu.sync_copy(y4_hbm.at[l, pl.ds(base, K)], y_v)
        m2cp = pltpu.async_copy(m2_hbm.at[pl.ds(base, K)], m2_v, sem2)
        pltpu.async_copy(hn_hbm.at[src_v], rows_v, sem).wait()
        m2cp.wait()

        def edge16(i, _):
          yv = y_v[pl.ds(i * 16, 16)]
          for e in range(16):
            row = i * 16 + e
            ye = yv[e]
            for jj in range(HID // 16):
              sl = pl.ds(jj * 16, 16)
              rows_v[row, sl] = rows_v[row, sl] * ye + m2_v[row, sl]
          return 0

        lax.fori_loop(0, K // 16, edge16, 0)
        pltpu.sync_copy(rows_v, acc_sh.at[dst_v], add=True)

      return 0

    lax.fori_loop(0, CPW, chunk, 0)
    plsc.subcore_barrier()
    pltpu.sync_copy(acc_sh.at[pl.ds(sid * 624, 624)],
                    out_hbm.at[cid, pl.ds(sid * 624, 624)])

    @pl.when(sid == 0)
    def _():
      pltpu.sync_copy(acc_sh.at[pl.ds(16 * 624, 16)],
                      out_hbm.at[cid, pl.ds(16 * 624, 16)])

  return _sc_spmm


_SPMMS = [_make_spmm(l) for l in range(NLAYERS)]


_EB = 2560
_NB = 1000


def _tc_y_body(eattr_ref, wemb_ref, bemb_ref, a4_ref, d_ref, y_ref, y4t_ref):
  ea = jnp.dot(eattr_ref[...], wemb_ref[...],
               preferred_element_type=jnp.float32) + bemb_ref[...]
  cols = []
  for l in range(NLAYERS):
    z = jnp.dot(ea, a4_ref[...][:, l:l + 1],
                preferred_element_type=jnp.float32)
    cols.append(jax.nn.sigmoid(z + d_ref[...][l:l + 1, :]))
  ycat = jnp.concatenate(cols, axis=1)
  y_ref[...] = ycat
  y4t_ref[...] = ycat.T


def _tc_y(edge_attr, Wemb, bemb, A4, dvec):
  return pl.pallas_call(
      _tc_y_body,
      grid=(E // _EB,),
      in_specs=[
          pl.BlockSpec((_EB, E_IN), lambda i: (i, 0)),
          pl.BlockSpec((E_IN, HID), lambda i: (0, 0)),
          pl.BlockSpec((1, HID), lambda i: (0, 0)),
          pl.BlockSpec((HID, NLAYERS), lambda i: (0, 0)),
          pl.BlockSpec((NLAYERS, 1), lambda i: (0, 0)),
      ],
      out_specs=[
          pl.BlockSpec((_EB, NLAYERS), lambda i: (i, 0)),
          pl.BlockSpec((NLAYERS, _EB), lambda i: (0, i)),
      ],
      out_shape=[
          jax.ShapeDtypeStruct((E, NLAYERS), jnp.float32),
          jax.ShapeDtypeStruct((NLAYERS, E), jnp.float32),
      ],
  )(edge_attr, Wemb, bemb.reshape(1, HID), A4, dvec.reshape(NLAYERS, 1))


def _tc_m2_body(l, eattr_ref, wemb_ref, bemb_ref, we_ref, be_ref, y_ref,
                m2_ref):
  ea = jnp.dot(eattr_ref[...], wemb_ref[...],
               preferred_element_type=jnp.float32) + bemb_ref[...]
  eemb = jnp.dot(ea, we_ref[...],
                 preferred_element_type=jnp.float32) + be_ref[...]
  m2_ref[...] = eemb * y_ref[...][:, l:l + 1]


def _tc_m2(l, edge_attr, Wemb, bemb, We, be, Yt):
  return pl.pallas_call(
      functools.partial(_tc_m2_body, l),
      grid=(E // _EB,),
      in_specs=[
          pl.BlockSpec((_EB, E_IN), lambda i: (i, 0)),
          pl.BlockSpec((E_IN, HID), lambda i: (0, 0)),
          pl.BlockSpec((1, HID), lambda i: (0, 0)),
          pl.BlockSpec((HID, HID), lambda i: (0, 0)),
          pl.BlockSpec((1, HID), lambda i: (0, 0)),
          pl.BlockSpec((_EB, NLAYERS), lambda i: (i, 0)),
      ],
      out_specs=pl.BlockSpec((_EB, HID), lambda i: (i, 0)),
      out_shape=jax.ShapeDtypeStruct((E, HID), jnp.float32),
  )(edge_attr, Wemb, bemb.reshape(1, HID), We, be.reshape(1, HID), Yt)


def _tc_embed_body(x_ref, w_ref, b_ref, h_ref):
  h_ref[...] = jnp.dot(x_ref[...], w_ref[...],
                       preferred_element_type=jnp.float32) + b_ref[...]


def _tc_embed(x, W, b):
  return pl.pallas_call(
      _tc_embed_body,
      grid=(N // _NB,),
      in_specs=[
          pl.BlockSpec((_NB, D_IN), lambda i: (i, 0)),
          pl.BlockSpec((D_IN, HID), lambda i: (0, 0)),
          pl.BlockSpec((1, HID), lambda i: (0, 0)),
      ],
      out_specs=pl.BlockSpec((_NB, HID), lambda i: (i, 0)),
      out_shape=jax.ShapeDtypeStruct((N, HID), jnp.float32),
  )(x, W, b.reshape(1, HID))


def _tc_layer_body(gp_ref, h_ref, g_ref, ho_ref):
  agg = gp_ref[0] + gp_ref[1]
  mu = jnp.mean(agg, axis=-1, keepdims=True)
  var = jnp.mean((agg - mu) ** 2, axis=-1, keepdims=True)
  ln = ((agg - mu) / jnp.sqrt(var + 1e-5)) * g_ref[...][0:1] + g_ref[...][1:2]
  ho_ref[...] = h_ref[...] + jnp.maximum(ln, 0.0)


def _tc_layer(Gp, h, ln_gb):
  return pl.pallas_call(
      _tc_layer_body,
      grid=(N // _NB,),
      in_specs=[
          pl.BlockSpec((NC, _NB, HID), lambda i: (0, i, 0)),
          pl.BlockSpec((_NB, HID), lambda i: (i, 0)),
          pl.BlockSpec((2, HID), lambda i: (0, 0)),
      ],
      out_specs=pl.BlockSpec((_NB, HID), lambda i: (i, 0)),
      out_shape=jax.ShapeDtypeStruct((N, HID), jnp.float32),
  )(Gp, h, ln_gb)


def _tc_head_body(h_ref, w1_ref, b1_ref, w2_ref, b2_ref, w3_ref, b3_ref,
                  o_ref):
  m = jnp.maximum(jnp.dot(h_ref[...], w1_ref[...],
                          preferred_element_type=jnp.float32) + b1_ref[...], 0.)
  m = jnp.maximum(jnp.dot(m, w2_ref[...],
                          preferred_element_type=jnp.float32) + b2_ref[...], 0.)
  o = jnp.dot(m, w3_ref[...], preferred_element_type=jnp.float32) + b3_ref[...]
  v = o[:, 0:1]
  s = o[:, 1:2]
  c = o[:, 2:3]
  norm = jnp.sqrt(s * s + c * c + 1e-8)
  o_ref[...] = jnp.concatenate(
      [v, s / norm, c / norm, o[:, 3:]], axis=1)


def _tc_head(h, W1, b1, W2, b2, W3, b3):
  return pl.pallas_call(
      _tc_head_body,
      grid=(N // _NB,),
      in_specs=[
          pl.BlockSpec((_NB, HID), lambda i: (i, 0)),
          pl.BlockSpec((HID, HID), lambda i: (0, 0)),
          pl.BlockSpec((1, HID), lambda i: (0, 0)),
          pl.BlockSpec((HID, HID // 2), lambda i: (0, 0)),
          pl.BlockSpec((1, HID // 2), lambda i: (0, 0)),
          pl.BlockSpec((HID // 2, 8), lambda i: (0, 0)),
          pl.BlockSpec((1, 8), lambda i: (0, 0)),
      ],
      out_specs=pl.BlockSpec((_NB, 8), lambda i: (i, 0)),
      out_shape=jax.ShapeDtypeStruct((N, 8), jnp.float32),
  )(h, W1, b1.reshape(1, HID), W2, b2.reshape(1, HID // 2), W3, b3)


def kernel(x, edge_index, edge_attr, params):
  p = params
  src = edge_index[0]
  dst = edge_index[1]
  Wemb, bemb = p['edge_embed']

  A4 = jnp.concatenate([lp['adm'][0] for lp in p['layers']], axis=1)
  dvec = jnp.stack([lp['adm'][1][0] for lp in p['layers']])
  ln_gbs = [jnp.stack([lp['ln'][0], lp['ln'][1]]) for lp in p['layers']]

  Yt, Y4T = _tc_y(edge_attr, Wemb, bemb, A4, dvec)
  M2s = [_tc_m2(l, edge_attr, Wemb, bemb, lp['lin_edge'][0],
                lp['lin_edge'][1], Yt) for l, lp in enumerate(p['layers'])]
  h = _tc_embed(x, p['node_embed'][0], p['node_embed'][1])

  for l, lp in enumerate(p['layers']):
    hn = _tc_embed(h, lp['lin_node'][0], lp['lin_node'][1])
    Gp = _SPMMS[l](hn, M2s[l], src, dst, Y4T)
    h = _tc_layer(Gp, h, ln_gbs[l])

  hp = p['head']
  W3 = jnp.concatenate([hp['v'][0], hp['s'][0], hp['c'][0],
                        jnp.zeros((HID // 2, 5), jnp.float32)], axis=1)
  b3 = jnp.concatenate([hp['v'][1], hp['s'][1], hp['c'][1],
                        jnp.zeros((5,), jnp.float32)]).reshape(1, 8)
  o = _tc_head(h, hp['mlp1'][0], hp['mlp1'][1], hp['mlp2'][0], hp['mlp2'][1],
               W3, b3)
  return (o[:, 0], o[:, 1], o[:, 2])


# speedup vs baseline: 1.3356x; 1.3356x over previous
import functools

import jax
import jax.numpy as jnp
from jax import lax
from jax.experimental import pallas as pl
from jax.experimental.pallas import tpu as pltpu
from jax.experimental.pallas import tpu_sc as plsc

N = 10000
E = 320000
D_IN = 128
E_IN = 16
HID = 128
NLAYERS = 4

NC = 2
NS = 16
NW = NC * NS
K = 128
NCHUNK = E // K
CPW = -(-NCHUNK // NW)
RPS = N // NS

_sc_mesh = plsc.VectorSubcoreMesh(
    core_axis_name="c", subcore_axis_name="s", num_cores=NC, num_subcores=NS)


def _zero_rows(rows_v, nrow, width):
  z = jnp.zeros((16,), jnp.float32)

  def body(i, _):
    for j in range(width // 16):
      rows_v[i, pl.ds(j * 16, 16)] = z
    return 0

  lax.fori_loop(0, nrow, body, 0, unroll=2)


def _make_spmm(l):
  @functools.partial(
      pl.kernel,
      out_type=jax.ShapeDtypeStruct((NC, N, HID), jnp.float32),
      mesh=_sc_mesh,
      scratch_types=[
          pltpu.VMEM((2, K), jnp.int32),
          pltpu.VMEM((K,), jnp.float32),
          pltpu.VMEM((K, HID), jnp.float32),
          pltpu.VMEM((K, HID), jnp.float32),
          pltpu.VMEM_SHARED((N, HID), jnp.float32),
          pltpu.SemaphoreType.DMA,
          pltpu.SemaphoreType.DMA,
      ],
  )
  def _sc_spmm(hn_hbm, m2_hbm, ei_hbm, y4_hbm, out_hbm,
               ei_v, y_v, rows_v, m2_v, acc_sh, sem, sem2):
    cid = lax.axis_index("c")
    sid = lax.axis_index("s")
    wid = sid * NC + cid

    _zero_rows(rows_v, K, HID)
    for t in range(4):
      pltpu.sync_copy(rows_v.at[pl.ds(0, K)],
                      acc_sh.at[pl.ds(sid * 624 + t * K, K)])
    pltpu.sync_copy(rows_v.at[pl.ds(0, 112)],
                    acc_sh.at[pl.ds(sid * 624 + 4 * K, 112)])

    @pl.when(sid == 0)
    def _():
      pltpu.sync_copy(rows_v.at[pl.ds(0, 16)], acc_sh.at[pl.ds(16 * 624, 16)])

    plsc.subcore_barrier()

    def chunk(j, _):
      c = wid + j * NW

      @pl.when(c < NCHUNK)
      def _():
        base = c * K
        pltpu.sync_copy(ei_hbm.at[pl.ds(0, 2), pl.ds(base, K)], ei_v)
        pltpu.sync_copy(y4_hbm.at[l, pl.ds(base, K)], y_v)
        m2cp = pltpu.async_copy(m2_hbm.at[pl.ds(base, K)], m2_v, sem2)
        pltpu.async_copy(hn_hbm.at[ei_v.at[0]], rows_v, sem).wait()

        def edge16(i, _):
          yv = y_v[pl.ds(i * 16, 16)]
          for e in range(16):
            row = i * 16 + e
            ye = yv[e]
            for jj in range(HID // 16):
              sl = pl.ds(jj * 16, 16)
              rows_v[row, sl] = rows_v[row, sl] * ye
          return 0

        lax.fori_loop(0, K // 16, edge16, 0)
        m2cp.wait()
        pltpu.sync_copy(rows_v, acc_sh.at[ei_v.at[1]], add=True)
        pltpu.sync_copy(m2_v, acc_sh.at[ei_v.at[1]], add=True)

      return 0

    lax.fori_loop(0, CPW, chunk, 0)
    plsc.subcore_barrier()
    pltpu.sync_copy(acc_sh.at[pl.ds(sid * 624, 624)],
                    out_hbm.at[cid, pl.ds(sid * 624, 624)])

    @pl.when(sid == 0)
    def _():
      pltpu.sync_copy(acc_sh.at[pl.ds(16 * 624, 16)],
                      out_hbm.at[cid, pl.ds(16 * 624, 16)])

  return _sc_spmm


_SPMMS = [_make_spmm(l) for l in range(NLAYERS)]


_EB = 2560
_NB = 1000


def _tc_y_body(eattr_ref, wemb_ref, bemb_ref, a4_ref, d_ref, y_ref, pk_ref):
  ea = jnp.dot(eattr_ref[...], wemb_ref[...],
               preferred_element_type=jnp.float32) + bemb_ref[...]
  cols = []
  for l in range(NLAYERS):
    z = jnp.dot(ea, a4_ref[...][:, l:l + 1],
                preferred_element_type=jnp.float32)
    cols.append(jax.nn.sigmoid(z + d_ref[...][l:l + 1, :]))
  ycat = jnp.concatenate(cols, axis=1)
  y_ref[...] = ycat
  pk_ref[...] = ycat.T


def _tc_y(edge_attr, Wemb, bemb, A4, dvec):
  return pl.pallas_call(
      _tc_y_body,
      grid=(E // _EB,),
      in_specs=[
          pl.BlockSpec((_EB, E_IN), lambda i: (i, 0)),
          pl.BlockSpec((E_IN, HID), lambda i: (0, 0)),
          pl.BlockSpec((1, HID), lambda i: (0, 0)),
          pl.BlockSpec((HID, NLAYERS), lambda i: (0, 0)),
          pl.BlockSpec((NLAYERS, 1), lambda i: (0, 0)),
      ],
      out_specs=[
          pl.BlockSpec((_EB, NLAYERS), lambda i: (i, 0)),
          pl.BlockSpec((NLAYERS, _EB), lambda i: (0, i)),
      ],
      out_shape=[
          jax.ShapeDtypeStruct((E, NLAYERS), jnp.float32),
          jax.ShapeDtypeStruct((NLAYERS, E), jnp.float32),
      ],
  )(edge_attr, Wemb, bemb.reshape(1, HID), A4, dvec.reshape(NLAYERS, 1))


def _tc_m2_body(l, eattr_ref, wemb_ref, bemb_ref, we_ref, be_ref, y_ref,
                m2_ref):
  ea = jnp.dot(eattr_ref[...], wemb_ref[...],
               preferred_element_type=jnp.float32) + bemb_ref[...]
  eemb = jnp.dot(ea, we_ref[...],
                 preferred_element_type=jnp.float32) + be_ref[...]
  m2_ref[...] = eemb * y_ref[...][:, l:l + 1]


def _tc_m2(l, edge_attr, Wemb, bemb, We, be, Yt):
  return pl.pallas_call(
      functools.partial(_tc_m2_body, l),
      grid=(E // _EB,),
      in_specs=[
          pl.BlockSpec((_EB, E_IN), lambda i: (i, 0)),
          pl.BlockSpec((E_IN, HID), lambda i: (0, 0)),
          pl.BlockSpec((1, HID), lambda i: (0, 0)),
          pl.BlockSpec((HID, HID), lambda i: (0, 0)),
          pl.BlockSpec((1, HID), lambda i: (0, 0)),
          pl.BlockSpec((_EB, NLAYERS), lambda i: (i, 0)),
      ],
      out_specs=pl.BlockSpec((_EB, HID), lambda i: (i, 0)),
      out_shape=jax.ShapeDtypeStruct((E, HID), jnp.float32),
  )(edge_attr, Wemb, bemb.reshape(1, HID), We, be.reshape(1, HID), Yt)


def _tc_embed_body(x_ref, w_ref, b_ref, h_ref):
  h_ref[...] = jnp.dot(x_ref[...], w_ref[...],
                       preferred_element_type=jnp.float32) + b_ref[...]


def _tc_embed(x, W, b):
  return pl.pallas_call(
      _tc_embed_body,
      grid=(N // _NB,),
      in_specs=[
          pl.BlockSpec((_NB, D_IN), lambda i: (i, 0)),
          pl.BlockSpec((D_IN, HID), lambda i: (0, 0)),
          pl.BlockSpec((1, HID), lambda i: (0, 0)),
      ],
      out_specs=pl.BlockSpec((_NB, HID), lambda i: (i, 0)),
      out_shape=jax.ShapeDtypeStruct((N, HID), jnp.float32),
  )(x, W, b.reshape(1, HID))


def _tc_layer_body(gp_ref, h_ref, g_ref, ho_ref):
  agg = gp_ref[0] + gp_ref[1]
  mu = jnp.mean(agg, axis=-1, keepdims=True)
  var = jnp.mean((agg - mu) ** 2, axis=-1, keepdims=True)
  ln = ((agg - mu) / jnp.sqrt(var + 1e-5)) * g_ref[...][0:1] + g_ref[...][1:2]
  ho_ref[...] = h_ref[...] + jnp.maximum(ln, 0.0)


def _tc_layer(Gp, h, ln_gb):
  return pl.pallas_call(
      _tc_layer_body,
      grid=(N // _NB,),
      in_specs=[
          pl.BlockSpec((NC, _NB, HID), lambda i: (0, i, 0)),
          pl.BlockSpec((_NB, HID), lambda i: (i, 0)),
          pl.BlockSpec((2, HID), lambda i: (0, 0)),
      ],
      out_specs=pl.BlockSpec((_NB, HID), lambda i: (i, 0)),
      out_shape=jax.ShapeDtypeStruct((N, HID), jnp.float32),
  )(Gp, h, ln_gb)


def _tc_head_body(h_ref, w1_ref, b1_ref, w2_ref, b2_ref, w3_ref, b3_ref,
                  o_ref):
  m = jnp.maximum(jnp.dot(h_ref[...], w1_ref[...],
                          preferred_element_type=jnp.float32) + b1_ref[...], 0.)
  m = jnp.maximum(jnp.dot(m, w2_ref[...],
                          preferred_element_type=jnp.float32) + b2_ref[...], 0.)
  o = jnp.dot(m, w3_ref[...], preferred_element_type=jnp.float32) + b3_ref[...]
  v = o[:, 0:1]
  s = o[:, 1:2]
  c = o[:, 2:3]
  norm = jnp.sqrt(s * s + c * c + 1e-8)
  o_ref[...] = jnp.concatenate(
      [v, s / norm, c / norm, o[:, 3:]], axis=1)


def _tc_head(h, W1, b1, W2, b2, W3, b3):
  return pl.pallas_call(
      _tc_head_body,
      grid=(N // _NB,),
      in_specs=[
          pl.BlockSpec((_NB, HID), lambda i: (i, 0)),
          pl.BlockSpec((HID, HID), lambda i: (0, 0)),
          pl.BlockSpec((1, HID), lambda i: (0, 0)),
          pl.BlockSpec((HID, HID // 2), lambda i: (0, 0)),
          pl.BlockSpec((1, HID // 2), lambda i: (0, 0)),
          pl.BlockSpec((HID // 2, 8), lambda i: (0, 0)),
          pl.BlockSpec((1, 8), lambda i: (0, 0)),
      ],
      out_specs=pl.BlockSpec((_NB, 8), lambda i: (i, 0)),
      out_shape=jax.ShapeDtypeStruct((N, 8), jnp.float32),
  )(h, W1, b1.reshape(1, HID), W2, b2.reshape(1, HID // 2), W3, b3)


def kernel(x, edge_index, edge_attr, params):
  p = params
  Wemb, bemb = p['edge_embed']

  A4 = jnp.concatenate([lp['adm'][0] for lp in p['layers']], axis=1)
  dvec = jnp.stack([lp['adm'][1][0] for lp in p['layers']])
  ln_gbs = [jnp.stack([lp['ln'][0], lp['ln'][1]]) for lp in p['layers']]

  Yt, Y4T = _tc_y(edge_attr, Wemb, bemb, A4, dvec)
  M2s = [_tc_m2(l, edge_attr, Wemb, bemb, lp['lin_edge'][0],
                lp['lin_edge'][1], Yt) for l, lp in enumerate(p['layers'])]
  h = _tc_embed(x, p['node_embed'][0], p['node_embed'][1])

  for l, lp in enumerate(p['layers']):
    hn = _tc_embed(h, lp['lin_node'][0], lp['lin_node'][1])
    Gp = _SPMMS[l](hn, M2s[l], edge_index, Y4T)
    h = _tc_layer(Gp, h, ln_gbs[l])

  hp = p['head']
  W3 = jnp.concatenate([hp['v'][0], hp['s'][0], hp['c'][0],
                        jnp.zeros((HID // 2, 5), jnp.float32)], axis=1)
  b3 = jnp.concatenate([hp['v'][1], hp['s'][1], hp['c'][1],
                        jnp.zeros((5,), jnp.float32)]).reshape(1, 8)
  o = _tc_head(h, hp['mlp1'][0], hp['mlp1'][1], hp['mlp2'][0], hp['mlp2'][1],
               W3, b3)
  return (o[:, 0], o[:, 1], o[:, 2])


# speedup vs baseline: 1.4296x; 1.0704x over previous
import functools

import jax
import jax.numpy as jnp
from jax import lax
from jax.experimental import pallas as pl
from jax.experimental.pallas import tpu as pltpu
from jax.experimental.pallas import tpu_sc as plsc

N = 10000
E = 320000
D_IN = 128
E_IN = 16
HID = 128
NLAYERS = 4

NC = 2
NS = 16
NW = NC * NS
K = 128
NCHUNK = E // K
CPW = -(-NCHUNK // NW)
RPS = N // NS

_sc_mesh = plsc.VectorSubcoreMesh(
    core_axis_name="c", subcore_axis_name="s", num_cores=NC, num_subcores=NS)


def _zero_rows(rows_v, nrow, width):
  z = jnp.zeros((16,), jnp.float32)

  def body(i, _):
    for j in range(width // 16):
      rows_v[i, pl.ds(j * 16, 16)] = z
    return 0

  lax.fori_loop(0, nrow, body, 0, unroll=2)


def _make_spmm(l):
  @functools.partial(
      pl.kernel,
      out_type=jax.ShapeDtypeStruct((NC, N, HID), jnp.float32),
      mesh=_sc_mesh,
      scratch_types=[
          pltpu.VMEM((2, K), jnp.int32),
          pltpu.VMEM((K,), jnp.float32),
          pltpu.VMEM((K, HID), jnp.float32),
          pltpu.VMEM((K, HID), jnp.float32),
          pltpu.VMEM_SHARED((N, HID), jnp.float32),
          pltpu.SemaphoreType.DMA,
          pltpu.SemaphoreType.DMA,
          pltpu.SemaphoreType.DMA,
      ],
  )
  def _sc_spmm(hn_hbm, m2_hbm, ei_hbm, y4_hbm, out_hbm,
               ei_v, y_v, rows_v, m2_v, acc_sh, sem, sem2, sem3):
    cid = lax.axis_index("c")
    sid = lax.axis_index("s")
    wid = sid * NC + cid

    _zero_rows(rows_v, K, HID)
    for t in range(4):
      pltpu.sync_copy(rows_v.at[pl.ds(0, K)],
                      acc_sh.at[pl.ds(sid * 624 + t * K, K)])
    pltpu.sync_copy(rows_v.at[pl.ds(0, 112)],
                    acc_sh.at[pl.ds(sid * 624 + 4 * K, 112)])

    @pl.when(sid == 0)
    def _():
      pltpu.sync_copy(rows_v.at[pl.ds(0, 16)], acc_sh.at[pl.ds(16 * 624, 16)])

    plsc.subcore_barrier()

    def chunk(j, _):
      c = wid + j * NW

      @pl.when(c < NCHUNK)
      def _():
        base = c * K
        pltpu.sync_copy(ei_hbm.at[pl.ds(0, 2), pl.ds(base, K)], ei_v)
        ycp = pltpu.async_copy(y4_hbm.at[l, pl.ds(base, K)], y_v, sem3)
        m2cp = pltpu.async_copy(m2_hbm.at[pl.ds(base, K)], m2_v, sem2)
        pltpu.async_copy(hn_hbm.at[ei_v.at[0]], rows_v, sem).wait()
        ycp.wait()

        def edge16(i, _):
          yv = y_v[pl.ds(i * 16, 16)]
          for e in range(16):
            row = i * 16 + e
            ye = yv[e]
            for jj in range(HID // 16):
              sl = pl.ds(jj * 16, 16)
              rows_v[row, sl] = rows_v[row, sl] * ye
          return 0

        lax.fori_loop(0, K // 16, edge16, 0)
        m2cp.wait()
        pltpu.sync_copy(rows_v, acc_sh.at[ei_v.at[1]], add=True)
        pltpu.sync_copy(m2_v, acc_sh.at[ei_v.at[1]], add=True)

      return 0

    lax.fori_loop(0, CPW, chunk, 0)
    plsc.subcore_barrier()
    pltpu.sync_copy(acc_sh.at[pl.ds(sid * 624, 624)],
                    out_hbm.at[cid, pl.ds(sid * 624, 624)])

    @pl.when(sid == 0)
    def _():
      pltpu.sync_copy(acc_sh.at[pl.ds(16 * 624, 16)],
                      out_hbm.at[cid, pl.ds(16 * 624, 16)])

  return _sc_spmm


_SPMMS = [_make_spmm(l) for l in range(NLAYERS)]


_EB = 2560
_NB = 1000


def _tc_y_body(eattr_ref, wemb_ref, bemb_ref, a4_ref, d_ref, y_ref, pk_ref):
  ea = jnp.dot(eattr_ref[...], wemb_ref[...],
               preferred_element_type=jnp.float32) + bemb_ref[...]
  cols = []
  for l in range(NLAYERS):
    z = jnp.dot(ea, a4_ref[...][:, l:l + 1],
                preferred_element_type=jnp.float32)
    cols.append(jax.nn.sigmoid(z + d_ref[...][l:l + 1, :]))
  ycat = jnp.concatenate(cols, axis=1)
  y_ref[...] = ycat
  pk_ref[...] = ycat.T


def _tc_y(edge_attr, Wemb, bemb, A4, dvec):
  return pl.pallas_call(
      _tc_y_body,
      grid=(E // _EB,),
      in_specs=[
          pl.BlockSpec((_EB, E_IN), lambda i: (i, 0)),
          pl.BlockSpec((E_IN, HID), lambda i: (0, 0)),
          pl.BlockSpec((1, HID), lambda i: (0, 0)),
          pl.BlockSpec((HID, NLAYERS), lambda i: (0, 0)),
          pl.BlockSpec((NLAYERS, 1), lambda i: (0, 0)),
      ],
      out_specs=[
          pl.BlockSpec((_EB, NLAYERS), lambda i: (i, 0)),
          pl.BlockSpec((NLAYERS, _EB), lambda i: (0, i)),
      ],
      out_shape=[
          jax.ShapeDtypeStruct((E, NLAYERS), jnp.float32),
          jax.ShapeDtypeStruct((NLAYERS, E), jnp.float32),
      ],
  )(edge_attr, Wemb, bemb.reshape(1, HID), A4, dvec.reshape(NLAYERS, 1))


def _tc_m2_body(l, eattr_ref, wemb_ref, bemb_ref, we_ref, be_ref, y_ref,
                m2_ref):
  ea = jnp.dot(eattr_ref[...], wemb_ref[...],
               preferred_element_type=jnp.float32) + bemb_ref[...]
  eemb = jnp.dot(ea, we_ref[...],
                 preferred_element_type=jnp.float32) + be_ref[...]
  m2_ref[...] = eemb * y_ref[...][:, l:l + 1]


def _tc_m2(l, edge_attr, Wemb, bemb, We, be, Yt):
  return pl.pallas_call(
      functools.partial(_tc_m2_body, l),
      grid=(E // _EB,),
      in_specs=[
          pl.BlockSpec((_EB, E_IN), lambda i: (i, 0)),
          pl.BlockSpec((E_IN, HID), lambda i: (0, 0)),
          pl.BlockSpec((1, HID), lambda i: (0, 0)),
          pl.BlockSpec((HID, HID), lambda i: (0, 0)),
          pl.BlockSpec((1, HID), lambda i: (0, 0)),
          pl.BlockSpec((_EB, NLAYERS), lambda i: (i, 0)),
      ],
      out_specs=pl.BlockSpec((_EB, HID), lambda i: (i, 0)),
      out_shape=jax.ShapeDtypeStruct((E, HID), jnp.float32),
  )(edge_attr, Wemb, bemb.reshape(1, HID), We, be.reshape(1, HID), Yt)


def _tc_embed_body(x_ref, w_ref, b_ref, h_ref):
  h_ref[...] = jnp.dot(x_ref[...], w_ref[...],
                       preferred_element_type=jnp.float32) + b_ref[...]


def _tc_embed(x, W, b):
  return pl.pallas_call(
      _tc_embed_body,
      grid=(N // _NB,),
      in_specs=[
          pl.BlockSpec((_NB, D_IN), lambda i: (i, 0)),
          pl.BlockSpec((D_IN, HID), lambda i: (0, 0)),
          pl.BlockSpec((1, HID), lambda i: (0, 0)),
      ],
      out_specs=pl.BlockSpec((_NB, HID), lambda i: (i, 0)),
      out_shape=jax.ShapeDtypeStruct((N, HID), jnp.float32),
  )(x, W, b.reshape(1, HID))


def _tc_layer_body(gp_ref, h_ref, g_ref, ho_ref):
  agg = gp_ref[0] + gp_ref[1]
  mu = jnp.mean(agg, axis=-1, keepdims=True)
  var = jnp.mean((agg - mu) ** 2, axis=-1, keepdims=True)
  ln = ((agg - mu) / jnp.sqrt(var + 1e-5)) * g_ref[...][0:1] + g_ref[...][1:2]
  ho_ref[...] = h_ref[...] + jnp.maximum(ln, 0.0)


def _tc_layer(Gp, h, ln_gb):
  return pl.pallas_call(
      _tc_layer_body,
      grid=(N // _NB,),
      in_specs=[
          pl.BlockSpec((NC, _NB, HID), lambda i: (0, i, 0)),
          pl.BlockSpec((_NB, HID), lambda i: (i, 0)),
          pl.BlockSpec((2, HID), lambda i: (0, 0)),
      ],
      out_specs=pl.BlockSpec((_NB, HID), lambda i: (i, 0)),
      out_shape=jax.ShapeDtypeStruct((N, HID), jnp.float32),
  )(Gp, h, ln_gb)


def _tc_head_body(h_ref, w1_ref, b1_ref, w2_ref, b2_ref, w3_ref, b3_ref,
                  o_ref):
  m = jnp.maximum(jnp.dot(h_ref[...], w1_ref[...],
                          preferred_element_type=jnp.float32) + b1_ref[...], 0.)
  m = jnp.maximum(jnp.dot(m, w2_ref[...],
                          preferred_element_type=jnp.float32) + b2_ref[...], 0.)
  o = jnp.dot(m, w3_ref[...], preferred_element_type=jnp.float32) + b3_ref[...]
  v = o[:, 0:1]
  s = o[:, 1:2]
  c = o[:, 2:3]
  norm = jnp.sqrt(s * s + c * c + 1e-8)
  o_ref[...] = jnp.concatenate(
      [v, s / norm, c / norm, o[:, 3:]], axis=1)


def _tc_head(h, W1, b1, W2, b2, W3, b3):
  return pl.pallas_call(
      _tc_head_body,
      grid=(N // _NB,),
      in_specs=[
          pl.BlockSpec((_NB, HID), lambda i: (i, 0)),
          pl.BlockSpec((HID, HID), lambda i: (0, 0)),
          pl.BlockSpec((1, HID), lambda i: (0, 0)),
          pl.BlockSpec((HID, HID // 2), lambda i: (0, 0)),
          pl.BlockSpec((1, HID // 2), lambda i: (0, 0)),
          pl.BlockSpec((HID // 2, 8), lambda i: (0, 0)),
          pl.BlockSpec((1, 8), lambda i: (0, 0)),
      ],
      out_specs=pl.BlockSpec((_NB, 8), lambda i: (i, 0)),
      out_shape=jax.ShapeDtypeStruct((N, 8), jnp.float32),
  )(h, W1, b1.reshape(1, HID), W2, b2.reshape(1, HID // 2), W3, b3)


def kernel(x, edge_index, edge_attr, params):
  p = params
  Wemb, bemb = p['edge_embed']

  A4 = jnp.concatenate([lp['adm'][0] for lp in p['layers']], axis=1)
  dvec = jnp.stack([lp['adm'][1][0] for lp in p['layers']])
  ln_gbs = [jnp.stack([lp['ln'][0], lp['ln'][1]]) for lp in p['layers']]

  Yt, Y4T = _tc_y(edge_attr, Wemb, bemb, A4, dvec)
  M2s = [_tc_m2(l, edge_attr, Wemb, bemb, lp['lin_edge'][0],
                lp['lin_edge'][1], Yt) for l, lp in enumerate(p['layers'])]
  h = _tc_embed(x, p['node_embed'][0], p['node_embed'][1])

  for l, lp in enumerate(p['layers']):
    hn = _tc_embed(h, lp['lin_node'][0], lp['lin_node'][1])
    Gp = _SPMMS[l](hn, M2s[l], edge_index, Y4T)
    h = _tc_layer(Gp, h, ln_gbs[l])

  hp = p['head']
  W3 = jnp.concatenate([hp['v'][0], hp['s'][0], hp['c'][0],
                        jnp.zeros((HID // 2, 5), jnp.float32)], axis=1)
  b3 = jnp.concatenate([hp['v'][1], hp['s'][1], hp['c'][1],
                        jnp.zeros((5,), jnp.float32)]).reshape(1, 8)
  o = _tc_head(h, hp['mlp1'][0], hp['mlp1'][1], hp['mlp2'][0], hp['mlp2'][1],
               W3, b3)
  return (o[:, 0], o[:, 1], o[:, 2])
